# baseline (speedup 1.0000x reference)
import functools

import jax
import jax.numpy as jnp
from jax.experimental import pallas as pl
from jax.experimental.pallas import tpu as pltpu

N_TOK = 32768
D = 768
N_EXP = 64
BLK = 512
NBUF = 16

NEG_INF = float("-inf")


def _top2(logits, vals_ref, idx_ref):
    col = jax.lax.broadcasted_iota(
        jnp.int32, logits.shape, 1).astype(jnp.float32)
    m1 = jnp.max(logits, axis=1, keepdims=True)
    i1 = jnp.min(jnp.where(logits == m1, col, float(N_EXP)), axis=1, keepdims=True)
    masked = jnp.where(col == i1, NEG_INF, logits)
    m2 = jnp.max(masked, axis=1, keepdims=True)
    i2 = jnp.min(jnp.where(masked == m2, col, float(N_EXP)), axis=1, keepdims=True)
    vals_ref[...] = jnp.concatenate([m1, m2], axis=1)
    idx_ref[...] = jnp.concatenate([i1, i2], axis=1).astype(jnp.int32)


def _router_kernel(x_hbm, w_ref, vals_ref, idx_ref, buf, sems):
    i = pl.program_id(0)
    n = pl.num_programs(0)

    def start_copy(c):
        slot = jax.lax.rem(c, NBUF)
        pltpu.make_async_copy(
            x_hbm.at[pl.ds(c * BLK, BLK), :],
            buf.at[slot],
            sems.at[slot],
        ).start()

    @pl.when(i == 0)
    def _():
        for j in range(NBUF):
            start_copy(jnp.int32(j))

    slot = jax.lax.rem(i, NBUF)
    pltpu.make_async_copy(
        x_hbm.at[pl.ds(i * BLK, BLK), :], buf.at[slot], sems.at[slot]
    ).wait()

    x = buf[slot]
    w = w_ref[...]
    logits = jax.lax.dot_general(
        x, w,
        dimension_numbers=(((1,), (1,)), ((), ())),
        preferred_element_type=jnp.float32,
    )
    _top2(logits, vals_ref, idx_ref)

    @pl.when(i + NBUF < n)
    def _():
        start_copy(i + NBUF)


@functools.partial(jax.jit, static_argnames=())
def kernel(x, w_g):
    grid = (N_TOK // BLK,)
    vals, idx = pl.pallas_call(
        _router_kernel,
        grid=grid,
        in_specs=[
            pl.BlockSpec(memory_space=pltpu.MemorySpace.HBM),
            pl.BlockSpec((N_EXP, D), lambda i: (0, 0)),
        ],
        out_specs=[
            pl.BlockSpec((BLK, 2), lambda i: (i, 0)),
            pl.BlockSpec((BLK, 2), lambda i: (i, 0)),
        ],
        out_shape=[
            jax.ShapeDtypeStruct((N_TOK, 2), jnp.float32),
            jax.ShapeDtypeStruct((N_TOK, 2), jnp.int32),
        ],
        scratch_shapes=[
            pltpu.VMEM((NBUF, BLK, D), jnp.float32),
            pltpu.SemaphoreType.DMA((NBUF,)),
        ],
        compiler_params=pltpu.CompilerParams(
            dimension_semantics=("arbitrary",),
        ),
    )(x, w_g)
    return (vals, idx)


# speedup vs baseline: 1.1352x; 1.1352x over previous
import functools

import jax
import jax.numpy as jnp
from jax.experimental import pallas as pl
from jax.experimental.pallas import tpu as pltpu

N_TOK = 32768
D = 768
N_EXP = 64
BLK = 1024
NBUF = 12

NEG_INF = float("-inf")


def _top2(logits, vals_ref, idx_ref):
    col = jax.lax.broadcasted_iota(
        jnp.int32, logits.shape, 1).astype(jnp.float32)
    m1 = jnp.max(logits, axis=1, keepdims=True)
    i1 = jnp.min(jnp.where(logits == m1, col, float(N_EXP)), axis=1, keepdims=True)
    masked = jnp.where(col == i1, NEG_INF, logits)
    m2 = jnp.max(masked, axis=1, keepdims=True)
    i2 = jnp.min(jnp.where(masked == m2, col, float(N_EXP)), axis=1, keepdims=True)
    vals_ref[...] = jnp.concatenate([m1, m2], axis=1)
    idx_ref[...] = jnp.concatenate([i1, i2], axis=1).astype(jnp.int32)


def _router_kernel(x_hbm, w_ref, vals_ref, idx_ref, buf, sems):
    i = pl.program_id(0)
    n = pl.num_programs(0)

    def start_copy(c):
        slot = jax.lax.rem(c, NBUF)
        pltpu.make_async_copy(
            x_hbm.at[pl.ds(c * BLK, BLK), :],
            buf.at[slot],
            sems.at[slot],
        ).start()

    @pl.when(i == 0)
    def _():
        for j in range(NBUF):
            start_copy(jnp.int32(j))

    slot = jax.lax.rem(i, NBUF)
    pltpu.make_async_copy(
        x_hbm.at[pl.ds(i * BLK, BLK), :], buf.at[slot], sems.at[slot]
    ).wait()

    x = buf[slot]
    w = w_ref[...]
    logits = jax.lax.dot_general(
        x, w,
        dimension_numbers=(((1,), (1,)), ((), ())),
        preferred_element_type=jnp.float32,
    )
    _top2(logits, vals_ref, idx_ref)

    @pl.when(i + NBUF < n)
    def _():
        start_copy(i + NBUF)


@functools.partial(jax.jit, static_argnames=())
def kernel(x, w_g):
    grid = (N_TOK // BLK,)
    vals, idx = pl.pallas_call(
        _router_kernel,
        grid=grid,
        in_specs=[
            pl.BlockSpec(memory_space=pltpu.MemorySpace.HBM),
            pl.BlockSpec((N_EXP, D), lambda i: (0, 0)),
        ],
        out_specs=[
            pl.BlockSpec((BLK, 2), lambda i: (i, 0)),
            pl.BlockSpec((BLK, 2), lambda i: (i, 0)),
        ],
        out_shape=[
            jax.ShapeDtypeStruct((N_TOK, 2), jnp.float32),
            jax.ShapeDtypeStruct((N_TOK, 2), jnp.int32),
        ],
        scratch_shapes=[
            pltpu.VMEM((NBUF, BLK, D), jnp.float32),
            pltpu.SemaphoreType.DMA((NBUF,)),
        ],
        compiler_params=pltpu.CompilerParams(
            dimension_semantics=("arbitrary",),
        ),
    )(x, w_g)
    return (vals, idx)


# speedup vs baseline: 1.1369x; 1.0014x over previous
import functools

import jax
import jax.numpy as jnp
from jax.experimental import pallas as pl
from jax.experimental.pallas import tpu as pltpu

N_TOK = 32768
D = 768
N_EXP = 64
BLK = 1024
NBUF = 5

NEG_INF = float("-inf")


def _top2(logits, vals_ref, idx_ref):
    col = jax.lax.broadcasted_iota(
        jnp.int32, logits.shape, 1).astype(jnp.float32)
    m1 = jnp.max(logits, axis=1, keepdims=True)
    i1 = jnp.min(jnp.where(logits == m1, col, float(N_EXP)), axis=1, keepdims=True)
    masked = jnp.where(col == i1, NEG_INF, logits)
    m2 = jnp.max(masked, axis=1, keepdims=True)
    i2 = jnp.min(jnp.where(masked == m2, col, float(N_EXP)), axis=1, keepdims=True)
    vals_ref[...] = jnp.concatenate([m1, m2], axis=1)
    idx_ref[...] = jnp.concatenate([i1, i2], axis=1).astype(jnp.int32)


def _router_kernel(x_hbm, w_ref, vals_ref, idx_ref, buf, sems):
    i = pl.program_id(0)
    n = pl.num_programs(0)

    def start_copy(c):
        slot = jax.lax.rem(c, NBUF)
        pltpu.make_async_copy(
            x_hbm.at[pl.ds(c * BLK, BLK), :],
            buf.at[slot],
            sems.at[slot],
        ).start()

    @pl.when(i == 0)
    def _():
        for j in range(NBUF):
            start_copy(jnp.int32(j))

    slot = jax.lax.rem(i, NBUF)
    pltpu.make_async_copy(
        x_hbm.at[pl.ds(i * BLK, BLK), :], buf.at[slot], sems.at[slot]
    ).wait()

    x = buf[slot]
    w = w_ref[...]
    logits = jax.lax.dot_general(
        x, w,
        dimension_numbers=(((1,), (1,)), ((), ())),
        preferred_element_type=jnp.float32,
    )
    _top2(logits, vals_ref, idx_ref)

    @pl.when(i + NBUF < n)
    def _():
        start_copy(i + NBUF)


@functools.partial(jax.jit, static_argnames=())
def kernel(x, w_g):
    grid = (N_TOK // BLK,)
    vals, idx = pl.pallas_call(
        _router_kernel,
        grid=grid,
        in_specs=[
            pl.BlockSpec(memory_space=pltpu.MemorySpace.HBM),
            pl.BlockSpec((N_EXP, D), lambda i: (0, 0)),
        ],
        out_specs=[
            pl.BlockSpec((BLK, 2), lambda i: (i, 0)),
            pl.BlockSpec((BLK, 2), lambda i: (i, 0)),
        ],
        out_shape=[
            jax.ShapeDtypeStruct((N_TOK, 2), jnp.float32),
            jax.ShapeDtypeStruct((N_TOK, 2), jnp.int32),
        ],
        scratch_shapes=[
            pltpu.VMEM((NBUF, BLK, D), jnp.float32),
            pltpu.SemaphoreType.DMA((NBUF,)),
        ],
        compiler_params=pltpu.CompilerParams(
            dimension_semantics=("arbitrary",),
        ),
    )(x, w_g)
    return (vals, idx)
